# baseline (speedup 1.0000x reference)
import functools

import jax
import jax.numpy as jnp
from jax import lax
from jax.experimental import pallas as pl
from jax.experimental.pallas import tpu as pltpu
from jax.experimental.pallas import tpu_sc as plsc

_N = 10000
_NPAD = 10240
_E = 320000
_NC, _NS = 2, 16
_NW = _NC * _NS
_K = 128
_EP = 327680
_EPW = _EP // _NW
_CH = _EPW // _K
_CHD = 2 * _CH
_D = 16
_RS = _NPAD // _NS
_G = 8
_NB = 4
_L = 16
_U = 8

_mesh = plsc.VectorSubcoreMesh(
    core_axis_name="c", subcore_axis_name="s",
    num_cores=_NC, num_subcores=_NS)
_SC_PARAMS = pltpu.CompilerParams(use_tc_tiling_on_sc=False,
                                 needs_layout_passes=False)


def _splat(v, u):
    return lax.gather(
        v, jnp.full((_L, 1), u, jnp.int32),
        lax.GatherDimensionNumbers(offset_dims=(), collapsed_slice_dims=(0,),
                                   start_index_map=(0,)),
        (1,), mode=lax.GatherScatterMode.PROMISE_IN_BOUNDS)


def _fast_rsqrt(d):
    xhalf = d * 0.5
    i = plsc.bitcast(d, jnp.int32)
    i = jnp.int32(0x5F3759DF) - lax.shift_right_logical(i, 1)
    y = plsc.bitcast(i, jnp.float32)
    y = y * (1.5 - xhalf * y * y)
    y = y * (1.5 - xhalf * y * y)
    y = y * (1.5 - xhalf * y * y)
    return jnp.where(d > 0.5, y, 0.0)


@functools.partial(
    pl.kernel,
    out_type=jax.ShapeDtypeStruct((_NPAD,), jnp.float32),
    mesh=_mesh, compiler_params=_SC_PARAMS,
    scratch_types=[
        pltpu.VMEM((_CHD, _K), jnp.int32),
        pltpu.VMEM((_K,), jnp.float32),
        pltpu.VMEM((_RS,), jnp.float32),
        pltpu.VMEM_SHARED((_NPAD,), jnp.float32),
        pltpu.SemaphoreType.DMA,
    ],
)
def _sc_degree(ei_hbm, dinv_hbm, idx_v, ones_v, dbuf, deg_sp, sem):
    c = lax.axis_index("c")
    s = lax.axis_index("s")
    one = jnp.full((_L,), 1.0, jnp.float32)
    zero = jnp.zeros((_L,), jnp.float32)
    for i in range(_K // _L):
        ones_v[pl.ds(i * _L, _L)] = one
    for i in range(_RS // _L):
        dbuf[pl.ds(i * _L, _L)] = zero
    pltpu.sync_copy(dbuf, deg_sp.at[pl.ds(s * _RS, _RS)])
    pltpu.sync_copy(ei_hbm.at[1, 2 * s], idx_v.at[pl.ds(0, _CH)])
    pltpu.sync_copy(ei_hbm.at[1, 2 * s + 1], idx_v.at[pl.ds(_CH, _CH)])
    plsc.subcore_barrier()

    def body(g, carry):
        for b in range(_G):
            pltpu.async_copy(ones_v, deg_sp.at[idx_v.at[g * _G + b]], sem,
                             add=True)
        for b in range(_G):
            pltpu.make_async_copy(ones_v, deg_sp.at[idx_v.at[g * _G + b]],
                                  sem).wait()
        return carry

    lax.fori_loop(0, _CHD // _G, body, 0)
    plsc.subcore_barrier()
    pltpu.sync_copy(deg_sp.at[pl.ds(s * _RS, _RS)], dbuf)

    def grp(g, carry):
        d = dbuf[pl.ds(g * _L, _L)]
        dbuf[pl.ds(g * _L, _L)] = _fast_rsqrt(d)
        return carry

    lax.fori_loop(0, _RS // _L, grp, 0)

    @pl.when(c == 0)
    def _():
        pltpu.sync_copy(dbuf, dinv_hbm.at[pl.ds(s * _RS, _RS)])


def _make_segsum(mode):
    @functools.partial(
        pl.kernel,
        out_type=jax.ShapeDtypeStruct((_NC, _NPAD, _D), jnp.float32),
        mesh=_mesh, compiler_params=_SC_PARAMS,
        scratch_types=(
            [pltpu.VMEM((_CH, _K), jnp.int32),
             pltpu.VMEM((_CH, _K), jnp.int32),
             pltpu.VMEM((_RS,), jnp.float32),
             pltpu.VMEM((_RS, _D), jnp.float32),
             pltpu.VMEM((_RS, _D), jnp.float32),
             pltpu.VMEM((_D,), jnp.float32)]
            + [pltpu.VMEM((_K, _D), jnp.float32)] * _NB
            + [pltpu.VMEM_SHARED((_NPAD, _D), jnp.float32),
               pltpu.VMEM_SHARED((_NPAD, _D), jnp.float32)]
            + [pltpu.SemaphoreType.DMA] * (2 * _NB)
        ),
    )
    def _seg(h_hbm, ei_hbm, dinv_hbm, bias_hbm, out_hbm, idx_s_v, idx_d_v,
             dbuf, hbuf, h2buf, bvec, r0, r1, r2, r3, g_sp, acc_sp,
             g0, g1_, g2_, g3, s0, s1, s2, s3):
        c = lax.axis_index("c")
        s = lax.axis_index("s")
        wid = s * _NC + c
        rows = (r0, r1, r2, r3)
        gsem = (g0, g1_, g2_, g3)
        ssem = (s0, s1, s2, s3)
        base = s * _RS
        zero = jnp.zeros((_L,), jnp.float32)
        for i in range(_K):
            r0[i] = zero
        for i in range(_RS // _K):
            pltpu.sync_copy(r0, acc_sp.at[pl.ds(base + i * _K, _K)])
        pltpu.sync_copy(dinv_hbm.at[pl.ds(base, _RS)], dbuf)
        pltpu.sync_copy(bias_hbm, bvec)
        if mode == 0:
            pltpu.sync_copy(h_hbm.at[pl.ds(base, _RS)], hbuf)
        else:
            pltpu.sync_copy(h_hbm.at[0, pl.ds(base, _RS)], hbuf)
            pltpu.sync_copy(h_hbm.at[1, pl.ds(base, _RS)], h2buf)
        bv = bvec[...]

        def stage(j, carry):
            dv = dbuf[pl.ds(j * _L, _L)]
            for u in range(_L):
                r = j * _L + u
                sc = _splat(dv, u)
                if mode == 0:
                    row = hbuf[r] * sc
                else:
                    t = sc * (hbuf[r] + h2buf[r]) + bv
                    row = sc * jnp.maximum(t, 0.0)
                hbuf[r] = row
            return carry

        lax.fori_loop(0, _RS // _L, stage, 0)
        pltpu.sync_copy(hbuf, g_sp.at[pl.ds(base, _RS)])
        pltpu.sync_copy(ei_hbm.at[0, wid], idx_s_v)
        pltpu.sync_copy(ei_hbm.at[1, wid], idx_d_v)
        plsc.subcore_barrier()

        def gather(ch, b):
            pltpu.async_copy(g_sp.at[idx_s_v.at[ch]], rows[b], gsem[b])

        def gather_wait(ch, b):
            pltpu.make_async_copy(g_sp.at[idx_s_v.at[ch]], rows[b],
                                  gsem[b]).wait()

        def scatter(ch, b):
            pltpu.async_copy(rows[b], acc_sp.at[idx_d_v.at[ch]], ssem[b],
                             add=True)

        def scatter_wait(ch, b):
            pltpu.make_async_copy(rows[b], acc_sp.at[idx_d_v.at[ch]],
                                  ssem[b]).wait()

        for b in range(_NB):
            gather(b, b)

        def body(p, carry):
            c0 = 4 * p
            for b in range(_NB):
                gather_wait(c0 + b, b)
                scatter(c0 + b, b)
            for b in range(_NB):
                n = c0 + 4 + b

                @pl.when(n < _CH)
                def _():
                    scatter_wait(c0 + b, b)
                    gather(n, b)

            return carry

        lax.fori_loop(0, _CH // _NB, body, 0)
        for b in range(_NB):
            scatter_wait(_CH - 4 + b, b)
        plsc.subcore_barrier()
        if mode == 0:
            pltpu.sync_copy(acc_sp.at[pl.ds(base, _RS)],
                            out_hbm.at[c, pl.ds(base, _RS)])
        else:
            pltpu.sync_copy(acc_sp.at[pl.ds(base, _RS)], hbuf)

            def post(j, carry):
                dv = dbuf[pl.ds(j * _L, _L)]
                for u in range(_L):
                    r = j * _L + u
                    sc = _splat(dv, u)
                    hbuf[r] = hbuf[r] * sc
                return carry

            lax.fori_loop(0, _RS // _L, post, 0)
            pltpu.sync_copy(hbuf, out_hbm.at[c, pl.ds(base, _RS)])

    return _seg


_sc_segsum1 = _make_segsum(0)
_sc_segsum2 = _make_segsum(1)


def _tc1_body(x_ref, w_ref, h_ref):
    h_ref[_N:, :] = jnp.zeros((_NPAD - _N, _D), jnp.float32)
    h_ref[:_N, :] = jnp.dot(x_ref[...], w_ref[...],
                            preferred_element_type=jnp.float32)


def _tc2_body(t_ref, w_ref, b_ref, o_ref):
    z = t_ref[0, :_N] + t_ref[1, :_N]
    o = jnp.dot(z, w_ref[...], preferred_element_type=jnp.float32) + b_ref[...]
    o = jnp.maximum(o, 0.0)
    m = jnp.max(o, axis=1, keepdims=True)
    sh = o - m
    lse = jnp.log(jnp.sum(jnp.exp(sh), axis=1, keepdims=True))
    o_ref[...] = sh - lse


def kernel(x, edge_index, W1, b1, W2, b2):
    f32 = jnp.float32
    ei3 = jnp.pad(edge_index.astype(jnp.int32), ((0, 0), (0, _EP - _E)),
                  constant_values=_NPAD - 1).reshape(2, _NW, _CH, _K)

    dinv = _sc_degree(ei3)

    h1 = pl.pallas_call(
        _tc1_body,
        out_shape=jax.ShapeDtypeStruct((_NPAD, _D), f32),
    )(x, W1)

    zero_bias = jnp.zeros((_D,), f32)
    acc1 = _sc_segsum1(h1, ei3, dinv, zero_bias)
    t2 = _sc_segsum2(acc1, ei3, dinv, b1)

    out = pl.pallas_call(
        _tc2_body,
        out_shape=jax.ShapeDtypeStruct((_N, W2.shape[1]), f32),
    )(t2, W2, b2.reshape(1, W2.shape[1]))

    return out


# speedup vs baseline: 1.1255x; 1.1255x over previous
import functools

import jax
import jax.numpy as jnp
from jax import lax
from jax.experimental import pallas as pl
from jax.experimental.pallas import tpu as pltpu
from jax.experimental.pallas import tpu_sc as plsc

_N = 10000
_NPAD = 10240
_E = 320000
_NC, _NS = 2, 16
_NW = _NC * _NS
_EPW = _E // _NW
_K = 80
_CH = _EPW // _K
_CHD = 2 * _CH
_D = 16
_RS = _NPAD // _NS
_G = 5
_NB = 4
_L = 16
_U = 8

_mesh = plsc.VectorSubcoreMesh(
    core_axis_name="c", subcore_axis_name="s",
    num_cores=_NC, num_subcores=_NS)
_SC_PARAMS = pltpu.CompilerParams(use_tc_tiling_on_sc=False,
                                 needs_layout_passes=False)


def _splat(v, u):
    return lax.gather(
        v, jnp.full((_L, 1), u, jnp.int32),
        lax.GatherDimensionNumbers(offset_dims=(), collapsed_slice_dims=(0,),
                                   start_index_map=(0,)),
        (1,), mode=lax.GatherScatterMode.PROMISE_IN_BOUNDS)


def _fast_rsqrt(d):
    xhalf = d * 0.5
    i = plsc.bitcast(d, jnp.int32)
    i = jnp.int32(0x5F3759DF) - lax.shift_right_logical(i, 1)
    y = plsc.bitcast(i, jnp.float32)
    y = y * (1.5 - xhalf * y * y)
    y = y * (1.5 - xhalf * y * y)
    y = y * (1.5 - xhalf * y * y)
    return jnp.where(d > 0.5, y, 0.0)


@functools.partial(
    pl.kernel,
    out_type=jax.ShapeDtypeStruct((_NPAD,), jnp.float32),
    mesh=_mesh, compiler_params=_SC_PARAMS,
    scratch_types=[
        pltpu.VMEM((_CHD, _K), jnp.int32),
        pltpu.VMEM((_K,), jnp.float32),
        pltpu.VMEM((_RS,), jnp.float32),
        pltpu.VMEM_SHARED((_NPAD,), jnp.float32),
        pltpu.SemaphoreType.DMA,
    ],
)
def _sc_degree(ei_hbm, dinv_hbm, idx_v, ones_v, dbuf, deg_sp, sem):
    c = lax.axis_index("c")
    s = lax.axis_index("s")
    one = jnp.full((_L,), 1.0, jnp.float32)
    zero = jnp.zeros((_L,), jnp.float32)
    for i in range(_K // _L):
        ones_v[pl.ds(i * _L, _L)] = one
    for i in range(_RS // _L):
        dbuf[pl.ds(i * _L, _L)] = zero
    pltpu.sync_copy(dbuf, deg_sp.at[pl.ds(s * _RS, _RS)])
    pltpu.sync_copy(ei_hbm.at[1, 2 * s], idx_v.at[pl.ds(0, _CH)])
    pltpu.sync_copy(ei_hbm.at[1, 2 * s + 1], idx_v.at[pl.ds(_CH, _CH)])
    plsc.subcore_barrier()

    def body(g, carry):
        for b in range(_G):
            pltpu.async_copy(ones_v, deg_sp.at[idx_v.at[g * _G + b]], sem,
                             add=True)
        for b in range(_G):
            pltpu.make_async_copy(ones_v, deg_sp.at[idx_v.at[g * _G + b]],
                                  sem).wait()
        return carry

    lax.fori_loop(0, _CHD // _G, body, 0)
    plsc.subcore_barrier()
    pltpu.sync_copy(deg_sp.at[pl.ds(s * _RS, _RS)], dbuf)

    def grp(g, carry):
        d = dbuf[pl.ds(g * _L, _L)]
        dbuf[pl.ds(g * _L, _L)] = _fast_rsqrt(d)
        return carry

    lax.fori_loop(0, _RS // _L, grp, 0)

    @pl.when(c == 0)
    def _():
        pltpu.sync_copy(dbuf, dinv_hbm.at[pl.ds(s * _RS, _RS)])


def _make_segsum(mode):
    @functools.partial(
        pl.kernel,
        out_type=jax.ShapeDtypeStruct((_NC, _NPAD, _D), jnp.float32),
        mesh=_mesh, compiler_params=_SC_PARAMS,
        scratch_types=(
            [pltpu.VMEM((_CH, _K), jnp.int32),
             pltpu.VMEM((_CH, _K), jnp.int32),
             pltpu.VMEM((_RS,), jnp.float32),
             pltpu.VMEM((_RS, _D), jnp.float32),
             pltpu.VMEM((_RS, _D), jnp.float32),
             pltpu.VMEM((_D,), jnp.float32)]
            + [pltpu.VMEM((_K, _D), jnp.float32)] * _NB
            + [pltpu.VMEM_SHARED((_NPAD, _D), jnp.float32),
               pltpu.VMEM_SHARED((_NPAD, _D), jnp.float32)]
            + [pltpu.SemaphoreType.DMA] * (2 * _NB)
        ),
    )
    def _seg(h_hbm, ei_hbm, dinv_hbm, bias_hbm, out_hbm, idx_s_v, idx_d_v,
             dbuf, hbuf, h2buf, bvec, r0, r1, r2, r3, g_sp, acc_sp,
             g0, g1_, g2_, g3, s0, s1, s2, s3):
        c = lax.axis_index("c")
        s = lax.axis_index("s")
        wid = s * _NC + c
        rows = (r0, r1, r2, r3)
        gsem = (g0, g1_, g2_, g3)
        ssem = (s0, s1, s2, s3)
        base = s * _RS
        zero = jnp.zeros((_L,), jnp.float32)
        for i in range(_K):
            r0[i] = zero
        for i in range(_RS // _K):
            pltpu.sync_copy(r0, acc_sp.at[pl.ds(base + i * _K, _K)])
        pltpu.sync_copy(dinv_hbm.at[pl.ds(base, _RS)], dbuf)
        pltpu.sync_copy(bias_hbm, bvec)
        if mode == 0:
            pltpu.sync_copy(h_hbm.at[pl.ds(base, _RS)], hbuf)
        else:
            pltpu.sync_copy(h_hbm.at[0, pl.ds(base, _RS)], hbuf)
            pltpu.sync_copy(h_hbm.at[1, pl.ds(base, _RS)], h2buf)
        bv = bvec[...]

        def stage(j, carry):
            dv = dbuf[pl.ds(j * _L, _L)]
            for u in range(_L):
                r = j * _L + u
                sc = _splat(dv, u)
                if mode == 0:
                    row = hbuf[r] * sc
                else:
                    t = sc * (hbuf[r] + h2buf[r]) + bv
                    row = sc * jnp.maximum(t, 0.0)
                hbuf[r] = row
            return carry

        lax.fori_loop(0, _RS // _L, stage, 0)
        pltpu.sync_copy(hbuf, g_sp.at[pl.ds(base, _RS)])
        pltpu.sync_copy(ei_hbm.at[0, wid], idx_s_v)
        pltpu.sync_copy(ei_hbm.at[1, wid], idx_d_v)
        plsc.subcore_barrier()

        def gather(ch, b):
            pltpu.async_copy(g_sp.at[idx_s_v.at[ch]], rows[b], gsem[b])

        def gather_wait(ch, b):
            pltpu.make_async_copy(g_sp.at[idx_s_v.at[ch]], rows[b],
                                  gsem[b]).wait()

        def scatter(ch, b):
            pltpu.async_copy(rows[b], acc_sp.at[idx_d_v.at[ch]], ssem[b],
                             add=True)

        def scatter_wait(ch, b):
            pltpu.make_async_copy(rows[b], acc_sp.at[idx_d_v.at[ch]],
                                  ssem[b]).wait()

        for b in range(_NB):
            gather(b, b)

        def body(p, carry):
            c0 = 4 * p
            for b in range(_NB):
                gather_wait(c0 + b, b)
                scatter(c0 + b, b)
            for b in range(_NB):
                n = c0 + 4 + b

                @pl.when(n < _CH)
                def _():
                    scatter_wait(c0 + b, b)
                    gather(n, b)

            return carry

        lax.fori_loop(0, _CH // _NB, body, 0)
        gather_wait(_CH - 1, 0)
        scatter(_CH - 1, 0)
        scatter_wait(_CH - 1, 0)
        for b in range(1, _NB):
            scatter_wait(_CH - 5 + b, b)
        plsc.subcore_barrier()
        if mode == 0:
            pltpu.sync_copy(acc_sp.at[pl.ds(base, _RS)],
                            out_hbm.at[c, pl.ds(base, _RS)])
        else:
            pltpu.sync_copy(acc_sp.at[pl.ds(base, _RS)], hbuf)

            def post(j, carry):
                dv = dbuf[pl.ds(j * _L, _L)]
                for u in range(_L):
                    r = j * _L + u
                    sc = _splat(dv, u)
                    hbuf[r] = hbuf[r] * sc
                return carry

            lax.fori_loop(0, _RS // _L, post, 0)
            pltpu.sync_copy(hbuf, out_hbm.at[c, pl.ds(base, _RS)])

    return _seg


_sc_segsum1 = _make_segsum(0)
_sc_segsum2 = _make_segsum(1)


def _tc1_body(x_ref, w_ref, h_ref):
    h_ref[_N:, :] = jnp.zeros((_NPAD - _N, _D), jnp.float32)
    h_ref[:_N, :] = jnp.dot(x_ref[...], w_ref[...],
                            preferred_element_type=jnp.float32)


def _tc2_body(t_ref, w_ref, b_ref, o_ref):
    z = t_ref[0, :_N] + t_ref[1, :_N]
    o = jnp.dot(z, w_ref[...], preferred_element_type=jnp.float32) + b_ref[...]
    o = jnp.maximum(o, 0.0)
    m = jnp.max(o, axis=1, keepdims=True)
    sh = o - m
    lse = jnp.log(jnp.sum(jnp.exp(sh), axis=1, keepdims=True))
    o_ref[...] = sh - lse


def kernel(x, edge_index, W1, b1, W2, b2):
    f32 = jnp.float32
    ei3 = edge_index.astype(jnp.int32).reshape(2, _NW, _CH, _K)

    dinv = _sc_degree(ei3)

    h1 = pl.pallas_call(
        _tc1_body,
        out_shape=jax.ShapeDtypeStruct((_NPAD, _D), f32),
    )(x, W1)

    zero_bias = jnp.zeros((_D,), f32)
    acc1 = _sc_segsum1(h1, ei3, dinv, zero_bias)
    t2 = _sc_segsum2(acc1, ei3, dinv, b1)

    out = pl.pallas_call(
        _tc2_body,
        out_shape=jax.ShapeDtypeStruct((_N, W2.shape[1]), f32),
    )(t2, W2, b2.reshape(1, W2.shape[1]))

    return out


# speedup vs baseline: 1.1969x; 1.0635x over previous
import functools

import jax
import jax.numpy as jnp
from jax import lax
from jax.experimental import pallas as pl
from jax.experimental.pallas import tpu as pltpu
from jax.experimental.pallas import tpu_sc as plsc

_N = 10000
_NPAD = 10240
_E = 320000
_NC, _NS = 2, 16
_NW = _NC * _NS
_EPW = _E // _NW
_K = 80
_CH = _EPW // _K
_CHD = 2 * _CH
_D = 16
_RS = _NPAD // _NS
_G = 5
_NB = 4
_L = 16
_U = 8

_mesh = plsc.VectorSubcoreMesh(
    core_axis_name="c", subcore_axis_name="s",
    num_cores=_NC, num_subcores=_NS)
_SC_PARAMS = pltpu.CompilerParams(use_tc_tiling_on_sc=False,
                                 needs_layout_passes=False)


def _splat(v, u):
    return lax.gather(
        v, jnp.full((_L, 1), u, jnp.int32),
        lax.GatherDimensionNumbers(offset_dims=(), collapsed_slice_dims=(0,),
                                   start_index_map=(0,)),
        (1,), mode=lax.GatherScatterMode.PROMISE_IN_BOUNDS)


def _fast_rsqrt(d):
    xhalf = d * 0.5
    i = plsc.bitcast(d, jnp.int32)
    i = jnp.int32(0x5F3759DF) - lax.shift_right_logical(i, 1)
    y = plsc.bitcast(i, jnp.float32)
    y = y * (1.5 - xhalf * y * y)
    y = y * (1.5 - xhalf * y * y)
    y = y * (1.5 - xhalf * y * y)
    return jnp.where(d > 0.5, y, 0.0)


@functools.partial(
    pl.kernel,
    out_type=jax.ShapeDtypeStruct((_NPAD,), jnp.float32),
    mesh=_mesh, compiler_params=_SC_PARAMS,
    scratch_types=[
        pltpu.VMEM((_CHD, _K), jnp.int32),
        pltpu.VMEM((_K,), jnp.float32),
        pltpu.VMEM((_RS,), jnp.float32),
        pltpu.VMEM_SHARED((_NPAD,), jnp.float32),
        pltpu.SemaphoreType.DMA,
    ],
)
def _sc_degree(ei_hbm, dinv_hbm, idx_v, ones_v, dbuf, deg_sp, sem):
    c = lax.axis_index("c")
    s = lax.axis_index("s")
    one = jnp.full((_L,), 1.0, jnp.float32)
    zero = jnp.zeros((_L,), jnp.float32)
    for i in range(_K // _L):
        ones_v[pl.ds(i * _L, _L)] = one
    for i in range(_RS // _L):
        dbuf[pl.ds(i * _L, _L)] = zero
    pltpu.sync_copy(dbuf, deg_sp.at[pl.ds(s * _RS, _RS)])
    pltpu.sync_copy(ei_hbm.at[1, 2 * s], idx_v.at[pl.ds(0, _CH)])
    pltpu.sync_copy(ei_hbm.at[1, 2 * s + 1], idx_v.at[pl.ds(_CH, _CH)])
    plsc.subcore_barrier()

    def body(g, carry):
        for b in range(_G):
            pltpu.async_copy(ones_v, deg_sp.at[idx_v.at[g * _G + b]], sem,
                             add=True)
        for b in range(_G):
            pltpu.make_async_copy(ones_v, deg_sp.at[idx_v.at[g * _G + b]],
                                  sem).wait()
        return carry

    lax.fori_loop(0, _CHD // _G, body, 0)
    plsc.subcore_barrier()
    pltpu.sync_copy(deg_sp.at[pl.ds(s * _RS, _RS)], dbuf)

    def grp(g, carry):
        d = dbuf[pl.ds(g * _L, _L)]
        dbuf[pl.ds(g * _L, _L)] = _fast_rsqrt(d)
        return carry

    lax.fori_loop(0, _RS // _L, grp, 0)

    @pl.when(c == 0)
    def _():
        pltpu.sync_copy(dbuf, dinv_hbm.at[pl.ds(s * _RS, _RS)])


def _make_segsum(mode):
    @functools.partial(
        pl.kernel,
        out_type=jax.ShapeDtypeStruct((_NC, _NPAD, _D), jnp.float32),
        mesh=_mesh, compiler_params=_SC_PARAMS,
        scratch_types=(
            [pltpu.VMEM((_CH, _K), jnp.int32),
             pltpu.VMEM((_CH, _K), jnp.int32),
             pltpu.VMEM((_RS,), jnp.float32),
             pltpu.VMEM((_RS, _D), jnp.float32),
             pltpu.VMEM((_RS, _D), jnp.float32),
             pltpu.VMEM((_D,), jnp.float32)]
            + [pltpu.VMEM((_K, _D), jnp.float32)] * _NB
            + [pltpu.VMEM_SHARED((_NPAD, _D), jnp.float32),
               pltpu.VMEM_SHARED((_NPAD, _D), jnp.float32)]
            + [pltpu.SemaphoreType.DMA] * (2 * _NB)
        ),
    )
    def _seg(h_hbm, ei_hbm, dinv_hbm, bias_hbm, out_hbm, idx_s_v, idx_d_v,
             dbuf, hbuf, h2buf, bvec, r0, r1, r2, r3, g_sp, acc_sp,
             g0, g1_, g2_, g3, s0, s1, s2, s3):
        c = lax.axis_index("c")
        s = lax.axis_index("s")
        wid = s * _NC + c
        rows = (r0, r1, r2, r3)
        gsem = (g0, g1_, g2_, g3)
        ssem = (s0, s1, s2, s3)
        base = s * _RS
        zero = jnp.zeros((_L,), jnp.float32)
        for i in range(_K):
            r0[i] = zero
        for i in range(_RS // _K):
            pltpu.sync_copy(r0, acc_sp.at[pl.ds(base + i * _K, _K)])
        pltpu.sync_copy(dinv_hbm.at[pl.ds(base, _RS)], dbuf)
        pltpu.sync_copy(bias_hbm, bvec)
        if mode == 0:
            pltpu.sync_copy(h_hbm.at[pl.ds(base, _RS)], hbuf)
        else:
            pltpu.sync_copy(h_hbm.at[0, pl.ds(base, _RS)], hbuf)
            pltpu.sync_copy(h_hbm.at[1, pl.ds(base, _RS)], h2buf)
        bv = bvec[...]

        def stage(j, carry):
            dv = dbuf[pl.ds(j * _L, _L)]
            for u in range(_L):
                r = j * _L + u
                sc = _splat(dv, u)
                if mode == 0:
                    row = hbuf[r] * sc
                else:
                    t = sc * (hbuf[r] + h2buf[r]) + bv
                    row = sc * jnp.maximum(t, 0.0)
                hbuf[r] = row
            return carry

        lax.fori_loop(0, _RS // _L, stage, 0)
        pltpu.sync_copy(hbuf, g_sp.at[pl.ds(base, _RS)])
        pltpu.sync_copy(ei_hbm.at[0, wid], idx_s_v)
        pltpu.sync_copy(ei_hbm.at[1, wid], idx_d_v)
        plsc.subcore_barrier()

        def gather(ch, b):
            pltpu.async_copy(g_sp.at[idx_s_v.at[ch]], rows[b], gsem[b])

        def gather_wait(ch, b):
            pltpu.make_async_copy(g_sp.at[idx_s_v.at[ch]], rows[b],
                                  gsem[b]).wait()

        def scatter(ch, b):
            pltpu.async_copy(rows[b], acc_sp.at[idx_d_v.at[ch]], ssem[b],
                             add=True)

        def scatter_wait(ch, b):
            pltpu.make_async_copy(rows[b], acc_sp.at[idx_d_v.at[ch]],
                                  ssem[b]).wait()

        for b in range(_NB):
            gather(b, b)

        def body(p, carry):
            c0 = 4 * p
            for b in range(_NB):
                gather_wait(c0 + b, b)
                scatter(c0 + b, b)
            for b in range(_NB):
                n = c0 + 4 + b

                @pl.when(n < _CH)
                def _():
                    scatter_wait(c0 + b, b)
                    gather(n, b)

            return carry

        lax.fori_loop(0, _CH // _NB, body, 0)
        gather_wait(_CH - 1, 0)
        scatter(_CH - 1, 0)
        scatter_wait(_CH - 1, 0)
        for b in range(1, _NB):
            scatter_wait(_CH - 5 + b, b)
        plsc.subcore_barrier()
        if mode == 0:
            pltpu.sync_copy(acc_sp.at[pl.ds(base, _RS)],
                            out_hbm.at[c, pl.ds(base, _RS)])
        else:
            pltpu.sync_copy(acc_sp.at[pl.ds(base, _RS)], hbuf)

            def post(j, carry):
                dv = dbuf[pl.ds(j * _L, _L)]
                for u in range(_L):
                    r = j * _L + u
                    sc = _splat(dv, u)
                    hbuf[r] = hbuf[r] * sc
                return carry

            lax.fori_loop(0, _RS // _L, post, 0)
            pltpu.sync_copy(hbuf, out_hbm.at[c, pl.ds(base, _RS)])

    return _seg


_sc_segsum1 = _make_segsum(0)
_sc_segsum2 = _make_segsum(1)


def _tc1_body(x_ref, w_ref, h_ref):
    h_ref[_N:, :] = jnp.zeros((_NPAD - _N, _D), jnp.float32)
    h_ref[:_N, :] = jnp.dot(x_ref[...], w_ref[...],
                            preferred_element_type=jnp.float32)


def _tc2_body(t_ref, w_ref, b_ref, bs_ref, bc_ref, o_ref):
    z = t_ref[0] + t_ref[1]
    o = jnp.dot(z, w_ref[...], preferred_element_type=jnp.float32) + b_ref[...]
    o = jnp.maximum(o, 0.0)
    m = jnp.max(o, axis=1, keepdims=True)
    sh = o - m
    e = jnp.exp(sh)
    s8 = jnp.dot(e, bs_ref[...], preferred_element_type=jnp.float32)
    lsb = jnp.log(s8)
    o_ref[...] = sh - jnp.dot(lsb, bc_ref[...],
                              preferred_element_type=jnp.float32)


def kernel(x, edge_index, W1, b1, W2, b2):
    f32 = jnp.float32
    ei3 = edge_index.astype(jnp.int32).reshape(2, _NW, _CH, _K)

    dinv = _sc_degree(ei3)

    h1 = pl.pallas_call(
        _tc1_body,
        out_shape=jax.ShapeDtypeStruct((_NPAD, _D), f32),
    )(x, W1)

    zero_bias = jnp.zeros((_D,), f32)
    acc1 = _sc_segsum1(h1, ei3, dinv, zero_bias)
    t2 = _sc_segsum2(acc1, ei3, dinv, b1)

    dout = W2.shape[1]
    eye8 = jnp.eye(8, dtype=f32)
    w2big = jnp.kron(eye8, W2)
    b2big = jnp.tile(b2, 8).reshape(1, 8 * dout)
    bsum = jnp.kron(eye8, jnp.ones((dout, 1), f32))
    bcast = jnp.kron(eye8, jnp.ones((1, dout), f32))
    o = pl.pallas_call(
        _tc2_body,
        out_shape=jax.ShapeDtypeStruct((_NPAD // 8, 8 * dout), f32),
    )(t2.reshape(_NC, _NPAD // 8, 128), w2big, b2big, bsum, bcast)

    return o.reshape(_NPAD, dout)[:_N]
